# baseline (speedup 1.0000x reference)
import functools

import jax
import jax.numpy as jnp
from jax import lax
from jax.experimental import pallas as pl
from jax.experimental.pallas import tpu as pltpu
from jax.experimental.pallas import tpu_sc as plsc

N_NODES = 10000
N_EDGES = 160000
IN_FEATS = 74
HIDDEN = 64
HEADS = 2
E_DIM = 11

NC = 2
NS = 16
NW = NC * NS
EDGES_PER_WORKER = N_EDGES // NW
CHUNK = 16
EPW_PAD = 5024
NCHUNK = EPW_PAD // CHUNK
D_NODE = HEADS * 3 * HIDDEN
D_SRC = D_NODE
D_QK = HEADS * 2 * HIDDEN
D_QKD = D_QK
D_OUT = 128
N_PAD = 10240
ROWS_PER_TILE = N_PAD // NS
F32 = jnp.float32


def _node_kernel(h_ref, w_ref, wqkv_ref, z_ref, zqk_ref):
    hb = h_ref[...]
    for hd in range(HEADS):
        t = lax.dot_general(hb, w_ref[hd], (((1,), (1,)), ((), ())),
                            preferred_element_type=F32)
        z = lax.dot_general(t, wqkv_ref[hd], (((1,), (1,)), ((), ())),
                            preferred_element_type=F32)
        z_ref[:, hd * 192:(hd + 1) * 192] = z
        zqk_ref[:, hd * 128:(hd + 1) * 128] = z[:, :128]


def _node_transform(hmat, w_heads, wqkv_heads):
    in_f = hmat.shape[1]
    bn = 2000
    return pl.pallas_call(
        _node_kernel,
        grid=(N_NODES // bn,),
        in_specs=[
            pl.BlockSpec((bn, in_f), lambda i: (i, 0)),
            pl.BlockSpec((HEADS, HIDDEN, in_f), lambda i: (0, 0, 0)),
            pl.BlockSpec((HEADS, 3 * HIDDEN, HIDDEN), lambda i: (0, 0, 0)),
        ],
        out_specs=[
            pl.BlockSpec((bn, D_SRC), lambda i: (i, 0)),
            pl.BlockSpec((bn, D_QKD), lambda i: (i, 0)),
        ],
        out_shape=[
            jax.ShapeDtypeStruct((N_NODES, D_SRC), F32),
            jax.ShapeDtypeStruct((N_NODES, D_QKD), F32),
        ],
    )(hmat, w_heads, wqkv_heads)


def _edge_kernel(ef_ref, wqkv_ref, we_ref, out_ref):
    ef = ef_ref[...]
    for hd in range(HEADS):
        m = lax.dot_general(wqkv_ref[hd], we_ref[hd],
                            (((1,), (0,)), ((), ())),
                            preferred_element_type=F32)
        out = lax.dot_general(ef, m, (((1,), (1,)), ((), ())),
                              preferred_element_type=F32)
        out_ref[:, hd * 192:(hd + 1) * 192] = out


def _edge_transform(e_f, wqkv_heads, we_heads):
    be = 4000
    return pl.pallas_call(
        _edge_kernel,
        grid=(N_EDGES // be,),
        in_specs=[
            pl.BlockSpec((be, E_DIM), lambda i: (i, 0)),
            pl.BlockSpec((HEADS, 3 * HIDDEN, HIDDEN), lambda i: (0, 0, 0)),
            pl.BlockSpec((HEADS, HIDDEN, E_DIM), lambda i: (0, 0, 0)),
        ],
        out_specs=pl.BlockSpec((be, D_SRC), lambda i: (i, 0)),
        out_shape=jax.ShapeDtypeStruct((N_EDGES, D_SRC), F32),
    )(e_f, wqkv_heads, we_heads)


def _sc_body(znode, zqk, eedge, idx4,
             out_m, out_s,
             idxq0, idxq1, isg0, isg1, idg0, idg1, ieg0, ieg1, iw0, iw1,
             iw30, iw31, sr0, sr1, dr0, dr1, er0, er1, mb0, mb1, s30, s31,
             zbuf, zbuf_c, acc_m, acc_s,
             g0, g1, sc0, sc1, ix0, ix1):
    c = lax.axis_index("c")
    s = lax.axis_index("s")

    idxq = (idxq0, idxq1)
    isg = (isg0, isg1)
    idg = (idg0, idg1)
    ieg = (ieg0, ieg1)
    iw = (iw0, iw1)
    iw3 = (iw30, iw31)
    srows = (sr0, sr1)
    drows = (dr0, dr1)
    erows = (er0, er1)
    mbuf = (mb0, mb1)
    s3 = (s30, s31)
    gsem = (g0, g1)
    scsem = (sc0, sc1)
    ixsem = (ix0, ix1)

    zero16 = jnp.zeros((16,), F32)
    one16 = jnp.ones((16,), F32)
    iota16 = lax.broadcasted_iota(jnp.int32, (16,), 0)
    for r in range(zbuf.shape[0]):
        for k in range(D_OUT // 16):
            zbuf[r, pl.ds(k * 16, 16)] = zero16
    for r in range(ROWS_PER_TILE // 16):
        zbuf_c[pl.ds(r * 16, 16)] = zero16
    s30[pl.ds(0, 16)] = one16
    s31[pl.ds(0, 16)] = one16

    row0 = s * ROWS_PER_TILE
    zb = zbuf.shape[0]
    for i in range(ROWS_PER_TILE // zb):
        pltpu.sync_copy(zbuf, acc_m.at[pl.ds(row0 + i * zb, zb)])
    for j in range(3):
        pltpu.sync_copy(zbuf_c, acc_s.at[pl.ds(j * N_PAD + row0,
                                               ROWS_PER_TILE)])
    plsc.subcore_barrier()

    bfly_idx = [jnp.bitwise_xor(iota16, sh) for sh in (8, 4, 2, 1)]
    gdn = lax.GatherDimensionNumbers(offset_dims=(), collapsed_slice_dims=(0,),
                                     start_index_map=(0,))

    def lanesum(v):
        for idx in bfly_idx:
            v = v + lax.gather(v, idx[:, None], gdn, slice_sizes=(1,),
                               mode=lax.GatherScatterMode.PROMISE_IN_BOUNDS)
        return v

    base4 = (c * NS + s) * (EPW_PAD * 4)

    def idx_args(t2, b):
        return (idx4.at[pl.ds(base4 + t2 * (CHUNK * 4), CHUNK * 4)],
                idxq[b], ixsem[b])

    def issue_idx(t2, b):
        pltpu.async_copy(*idx_args(t2, b))

    def wait_idx(t2, b):
        pltpu.make_async_copy(*idx_args(t2, b)).wait()

    def unpack_gidx(b):
        isg[b][pl.ds(0, 16)] = idxq[b][pl.ds(0, 16)]
        idg[b][pl.ds(0, 16)] = idxq[b][pl.ds(16, 16)]
        ieg[b][pl.ds(0, 16)] = idxq[b][pl.ds(32, 16)]

    def unpack_widx(b):
        v = idxq[b][pl.ds(48, 16)]
        iw[b][pl.ds(0, 16)] = v
        iw3[b][pl.ds(0, 16)] = v
        iw3[b][pl.ds(16, 16)] = v + N_PAD
        iw3[b][pl.ds(32, 16)] = v + 2 * N_PAD

    def gather_args(b):
        return ((znode.at[isg[b]], srows[b], gsem[b]),
                (zqk.at[idg[b]], drows[b], gsem[b]),
                (eedge.at[ieg[b]], erows[b], gsem[b]))

    def issue_gathers(b):
        for a in gather_args(b):
            pltpu.async_copy(*a)

    def wait_gathers(b):
        for a in gather_args(b):
            pltpu.make_async_copy(*a).wait()

    def scatter_args(b):
        return ((mbuf[b], acc_m.at[iw[b]], scsem[b]),
                (s3[b], acc_s.at[iw3[b]], scsem[b]))

    def issue_scatters(b):
        for a in scatter_args(b):
            pltpu.async_copy(*a, add=True)

    def wait_scatters(b):
        for a in scatter_args(b):
            pltpu.make_async_copy(*a).wait()

    def compute(b):
        sb, db, eb_, mb = srows[b], drows[b], erows[b], mbuf[b]

        def one_edge(e, carry2):
            c1s = []
            for hd in range(HEADS):
                o = hd * 192
                oq = hd * 128
                qs = [sb[e, pl.ds(o + k * 16, 16)] for k in range(4)]
                ks_ = [sb[e, pl.ds(o + 64 + k * 16, 16)] for k in range(4)]
                vs = [sb[e, pl.ds(o + 128 + k * 16, 16)] for k in range(4)]
                qd = [db[e, pl.ds(oq + k * 16, 16)] for k in range(4)]
                kd = [db[e, pl.ds(oq + 64 + k * 16, 16)] for k in range(4)]
                qe = [eb_[e, pl.ds(o + k * 16, 16)] for k in range(4)]
                ke = [eb_[e, pl.ds(o + 64 + k * 16, 16)] for k in range(4)]
                ve = [eb_[e, pl.ds(o + 128 + k * 16, 16)] for k in range(4)]

                def dotv(a, b2):
                    p = a[0] * b2[0]
                    for k in range(1, 4):
                        p = p + a[k] * b2[k]
                    return lanesum(p)

                lg = [[dotv(qs, ks_), dotv(qs, kd), dotv(qs, ke)],
                      [dotv(qd, ks_), dotv(qd, kd), dotv(qd, ke)],
                      [dotv(qe, ks_), dotv(qe, kd), dotv(qe, ke)]]
                col = [None, None, None]
                for i in range(3):
                    e0 = jnp.exp(lg[i][0])
                    e1 = jnp.exp(lg[i][1])
                    e2 = jnp.exp(lg[i][2])
                    rinv = 1.0 / (e0 + e1 + e2)
                    row = (e0 * rinv, e1 * rinv, e2 * rinv)
                    for j in range(3):
                        col[j] = row[j] if col[j] is None else col[j] + row[j]
                for k in range(4):
                    mk = col[0] * vs[k] + col[2] * ve[k]
                    mb[e, pl.ds(hd * 64 + k * 16, 16)] = mk
                c1s.append(col[1])
            sel = iota16 == jnp.full((16,), e, jnp.int32)
            c1v0 = jnp.where(sel, c1s[0], carry2[0])
            c1v1 = jnp.where(sel, c1s[1], carry2[1])
            return (c1v0, c1v1)

        final = plsc.parallel_loop(0, CHUNK, 1, unroll=2,
                                   carry=(zero16, zero16))(one_edge)
        s3[b][pl.ds(16, 16)] = final[0]
        s3[b][pl.ds(32, 16)] = final[1]

    issue_idx(0, 0)
    wait_idx(0, 0)
    unpack_gidx(0)
    unpack_widx(0)
    issue_idx(1, 1)
    wait_idx(1, 1)
    unpack_gidx(1)
    unpack_widx(1)
    issue_gathers(0)
    issue_gathers(1)
    wait_gathers(0)
    issue_idx(2, 0)
    compute(0)
    issue_scatters(0)
    wait_gathers(1)
    issue_idx(3, 1)
    wait_idx(2, 0)
    unpack_gidx(0)
    issue_gathers(0)
    compute(1)
    issue_scatters(1)

    def pair_body(u, carry):
        for b in (0, 1):
            t = 2 * u + b
            wait_gathers(b)
            wait_scatters(b)
            unpack_widx(b)
            issue_idx(t + 2, b)
            wait_idx(t + 1, 1 - b)
            unpack_gidx(1 - b)
            issue_gathers(1 - b)
            compute(b)
            issue_scatters(b)
        return carry

    lax.fori_loop(1, NCHUNK // 2, pair_body, 0)

    wait_scatters(0)
    wait_scatters(1)
    wait_gathers(0)
    wait_idx(NCHUNK + 1, 1)

    plsc.subcore_barrier()
    pltpu.sync_copy(acc_m.at[pl.ds(row0, ROWS_PER_TILE)],
                    out_m.at[c, pl.ds(row0, ROWS_PER_TILE)])
    pltpu.sync_copy(acc_s.at[pl.ds(s * (3 * ROWS_PER_TILE),
                                   3 * ROWS_PER_TILE)],
                    out_s.at[c, pl.ds(s * (3 * ROWS_PER_TILE),
                                      3 * ROWS_PER_TILE)])


_sc_conv = functools.partial(
    pl.kernel,
    out_type=(jax.ShapeDtypeStruct((NC, N_PAD, D_OUT), F32),
              jax.ShapeDtypeStruct((NC, 3 * N_PAD), F32)),
    mesh=plsc.VectorSubcoreMesh(core_axis_name="c", subcore_axis_name="s",
                                num_cores=NC, num_subcores=NS),
    scratch_types=[
        pltpu.VMEM((4 * CHUNK,), jnp.int32),
        pltpu.VMEM((4 * CHUNK,), jnp.int32),
        pltpu.VMEM((CHUNK,), jnp.int32),
        pltpu.VMEM((CHUNK,), jnp.int32),
        pltpu.VMEM((CHUNK,), jnp.int32),
        pltpu.VMEM((CHUNK,), jnp.int32),
        pltpu.VMEM((CHUNK,), jnp.int32),
        pltpu.VMEM((CHUNK,), jnp.int32),
        pltpu.VMEM((CHUNK,), jnp.int32),
        pltpu.VMEM((CHUNK,), jnp.int32),
        pltpu.VMEM((3 * CHUNK,), jnp.int32),
        pltpu.VMEM((3 * CHUNK,), jnp.int32),
        pltpu.VMEM((CHUNK, D_SRC), F32),
        pltpu.VMEM((CHUNK, D_SRC), F32),
        pltpu.VMEM((CHUNK, D_QKD), F32),
        pltpu.VMEM((CHUNK, D_QKD), F32),
        pltpu.VMEM((CHUNK, D_SRC), F32),
        pltpu.VMEM((CHUNK, D_SRC), F32),
        pltpu.VMEM((CHUNK, D_OUT), F32),
        pltpu.VMEM((CHUNK, D_OUT), F32),
        pltpu.VMEM((3 * CHUNK,), F32),
        pltpu.VMEM((3 * CHUNK,), F32),
        pltpu.VMEM((8, D_OUT), F32),
        pltpu.VMEM((ROWS_PER_TILE,), F32),
        pltpu.VMEM_SHARED((N_PAD, D_OUT), F32),
        pltpu.VMEM_SHARED((3 * N_PAD,), F32),
        pltpu.SemaphoreType.DMA,
        pltpu.SemaphoreType.DMA,
        pltpu.SemaphoreType.DMA,
        pltpu.SemaphoreType.DMA,
        pltpu.SemaphoreType.DMA,
        pltpu.SemaphoreType.DMA,
    ],
)(_sc_body)


def _epi_common(acc_ref, cnt_ref, w0_ref, w1_ref, z_ref,
                g_ref, b_ref, wout_ref, bout_ref):
    a = acc_ref[0] + acc_ref[1]
    cnt = cnt_ref[...]
    wh = (w0_ref[...], w1_ref[...])
    cols = []
    for hd in range(HEADS):
        zv = z_ref[:, hd * 192 + 128:hd * 192 + 192]
        m = a[:, hd * 64:(hd + 1) * 64] + wh[hd] * zv
        hn = jnp.where(cnt > 0, m / jnp.maximum(cnt, 1.0), 0.0)
        hn = jnp.maximum(hn, 0.0)
        mu = jnp.mean(hn, axis=-1, keepdims=True)
        var = jnp.mean((hn - mu) ** 2, axis=-1, keepdims=True)
        hn = (hn - mu) * lax.rsqrt(var + 1e-5) * g_ref[hd] + b_ref[hd]
        cols.append(hn)
    hc = jnp.concatenate(cols, axis=1)
    return lax.dot_general(hc, wout_ref[...], (((1,), (1,)), ((), ())),
                           preferred_element_type=F32) + bout_ref[...]


def _epi_kernel_mid(acc_ref, cnt_ref, w0_ref, w1_ref, z_ref,
                    g_ref, b_ref, wout_ref, bout_ref,
                    w1n_ref, wqkv1_ref, h1_ref, z1_ref, zqk1_ref):
    h1 = _epi_common(acc_ref, cnt_ref, w0_ref, w1_ref, z_ref,
                     g_ref, b_ref, wout_ref, bout_ref)
    h1_ref[...] = h1
    for hd in range(HEADS):
        t = lax.dot_general(h1, w1n_ref[hd], (((1,), (1,)), ((), ())),
                            preferred_element_type=F32)
        z = lax.dot_general(t, wqkv1_ref[hd], (((1,), (1,)), ((), ())),
                            preferred_element_type=F32)
        z1_ref[:, hd * 192:(hd + 1) * 192] = z
        zqk1_ref[:, hd * 128:(hd + 1) * 128] = z[:, :128]


def _epi_kernel_last(acc_ref, cnt_ref, w0_ref, w1_ref, z_ref,
                     g_ref, b_ref, wout_ref, bout_ref, h2_ref):
    h2_ref[...] = _epi_common(acc_ref, cnt_ref, w0_ref, w1_ref, z_ref,
                              g_ref, b_ref, wout_ref, bout_ref)


def _epi_in_specs(bn):
    return [
        pl.BlockSpec((NC, bn, D_OUT), lambda i: (0, i, 0)),
        pl.BlockSpec((bn, 1), lambda i: (i, 0)),
        pl.BlockSpec((bn, 1), lambda i: (i, 0)),
        pl.BlockSpec((bn, 1), lambda i: (i, 0)),
        pl.BlockSpec((bn, D_SRC), lambda i: (i, 0)),
        pl.BlockSpec((HEADS, HIDDEN), lambda i: (0, 0)),
        pl.BlockSpec((HEADS, HIDDEN), lambda i: (0, 0)),
        pl.BlockSpec((HIDDEN, HEADS * HIDDEN), lambda i: (0, 0)),
        pl.BlockSpec((HIDDEN,), lambda i: (0,)),
    ]


def _epilogue_mid(acc, cnt, w0, w1, z, g, b, wout, bout, w1n, wqkv1):
    bn = 2000
    return pl.pallas_call(
        _epi_kernel_mid,
        grid=(N_NODES // bn,),
        in_specs=_epi_in_specs(bn) + [
            pl.BlockSpec((HEADS, HIDDEN, HIDDEN), lambda i: (0, 0, 0)),
            pl.BlockSpec((HEADS, 3 * HIDDEN, HIDDEN), lambda i: (0, 0, 0)),
        ],
        out_specs=[
            pl.BlockSpec((bn, HIDDEN), lambda i: (i, 0)),
            pl.BlockSpec((bn, D_SRC), lambda i: (i, 0)),
            pl.BlockSpec((bn, D_QKD), lambda i: (i, 0)),
        ],
        out_shape=[
            jax.ShapeDtypeStruct((N_NODES, HIDDEN), F32),
            jax.ShapeDtypeStruct((N_NODES, D_SRC), F32),
            jax.ShapeDtypeStruct((N_NODES, D_QKD), F32),
        ],
    )(acc, cnt, w0, w1, z, g, b, wout, bout, w1n, wqkv1)


def _epilogue_last(acc, cnt, w0, w1, z, g, b, wout, bout):
    bn = 2000
    return pl.pallas_call(
        _epi_kernel_last,
        grid=(N_NODES // bn,),
        in_specs=_epi_in_specs(bn),
        out_specs=pl.BlockSpec((bn, HIDDEN), lambda i: (i, 0)),
        out_shape=jax.ShapeDtypeStruct((N_NODES, HIDDEN), F32),
    )(acc, cnt, w0, w1, z, g, b, wout, bout)


def _gru_cell(x, hprev, wih, whh, bih, bhh):
    gi = lax.dot_general(x, wih, (((1,), (1,)), ((), ())),
                         preferred_element_type=F32) + bih
    gh = lax.dot_general(hprev, whh, (((1,), (1,)), ((), ())),
                         preferred_element_type=F32) + bhh
    ir, iz, i_n = gi[:, :64], gi[:, 64:128], gi[:, 128:]
    hr, hz, h_n = gh[:, :64], gh[:, 64:128], gh[:, 128:]
    r = jax.nn.sigmoid(ir + hr)
    zz = jax.nn.sigmoid(iz + hz)
    nn_ = jnp.tanh(i_n + r * h_n)
    return (1.0 - zz) * nn_ + zz * hprev


def _gru_kernel(h_ref, h1_ref, h2_ref, pw_ref, pb_ref,
                wih0_ref, whh0_ref, bih0_ref, bhh0_ref,
                wih1_ref, whh1_ref, bih1_ref, bhh1_ref, out_ref):
    mh = jnp.mean(h_ref[...], axis=0, keepdims=True)
    x0 = lax.dot_general(mh, pw_ref[...], (((1,), (1,)), ((), ())),
                         preferred_element_type=F32) + pb_ref[...]
    x1 = jnp.mean(h1_ref[...], axis=0, keepdims=True)
    x2 = jnp.mean(h2_ref[...], axis=0, keepdims=True)
    seq = [x0, x1, x2]
    layers = [(wih0_ref, whh0_ref, bih0_ref, bhh0_ref),
              (wih1_ref, whh1_ref, bih1_ref, bhh1_ref)]
    finals = []
    for wih, whh, bih, bhh in layers:
        hf = jnp.zeros((1, HIDDEN), F32)
        outs_f = []
        for t in range(3):
            hf = _gru_cell(seq[t], hf, wih[0], whh[0], bih[0], bhh[0])
            outs_f.append(hf)
        hb = jnp.zeros((1, HIDDEN), F32)
        outs_b = [None, None, None]
        for t in (2, 1, 0):
            hb = _gru_cell(seq[t], hb, wih[1], whh[1], bih[1], bhh[1])
            outs_b[t] = hb
        finals += [hf, hb]
        seq = [jnp.concatenate([outs_f[t], outs_b[t]], axis=1) for t in range(3)]
    out_ref[...] = finals[0] + finals[1] + finals[2] + finals[3]


def _gru_readout(h, h1, h2, pw, pb, gru):
    wih0 = jnp.stack([gru[0]['f']['W_ih'], gru[0]['b']['W_ih']])
    whh0 = jnp.stack([gru[0]['f']['W_hh'], gru[0]['b']['W_hh']])
    bih0 = jnp.stack([gru[0]['f']['b_ih'], gru[0]['b']['b_ih']])
    bhh0 = jnp.stack([gru[0]['f']['b_hh'], gru[0]['b']['b_hh']])
    wih1 = jnp.stack([gru[1]['f']['W_ih'], gru[1]['b']['W_ih']])
    whh1 = jnp.stack([gru[1]['f']['W_hh'], gru[1]['b']['W_hh']])
    bih1 = jnp.stack([gru[1]['f']['b_ih'], gru[1]['b']['b_ih']])
    bhh1 = jnp.stack([gru[1]['f']['b_hh'], gru[1]['b']['b_hh']])
    whole = lambda *shape: pl.BlockSpec(shape, lambda: tuple(0 for _ in shape))
    return pl.pallas_call(
        _gru_kernel,
        in_specs=[
            whole(N_NODES, IN_FEATS),
            whole(N_NODES, HIDDEN),
            whole(N_NODES, HIDDEN),
            whole(HIDDEN, IN_FEATS),
            whole(HIDDEN,),
            whole(2, 3 * HIDDEN, HIDDEN),
            whole(2, 3 * HIDDEN, HIDDEN),
            whole(2, 3 * HIDDEN),
            whole(2, 3 * HIDDEN),
            whole(2, 3 * HIDDEN, 2 * HIDDEN),
            whole(2, 3 * HIDDEN, HIDDEN),
            whole(2, 3 * HIDDEN),
            whole(2, 3 * HIDDEN),
        ],
        out_specs=whole(1, HIDDEN),
        out_shape=jax.ShapeDtypeStruct((1, HIDDEN), F32),
    )(h, h1, h2, pw, pb, wih0, whh0, bih0, bhh0, wih1, whh1, bih1, bhh1)


def _col(x):
    return (x[0] + x[1])[:N_NODES].reshape(N_NODES, 1)


def kernel(h, e_f, edge_index, params):
    gat = params['gat']
    outs = params['out']

    w_heads = [jnp.stack([gat[cv][i]['W'] for i in range(HEADS)])
               for cv in range(2)]
    wqkv_heads = [
        jnp.stack([jnp.concatenate(
            [gat[cv][i]['Wq'], gat[cv][i]['Wk'], gat[cv][i]['Wv']], axis=0)
            for i in range(HEADS)])
        for cv in range(2)]
    g = [jnp.stack([gat[cv][i]['g'] for i in range(HEADS)]) for cv in range(2)]
    b = [jnp.stack([gat[cv][i]['b'] for i in range(HEADS)]) for cv in range(2)]

    pad = EPW_PAD - EDGES_PER_WORKER
    src = jnp.pad(edge_index[0].reshape(NW, EDGES_PER_WORKER),
                  ((0, 0), (0, pad))).reshape(-1, CHUNK)
    dst2 = edge_index[1].reshape(NW, EDGES_PER_WORKER)
    dst_g = jnp.pad(dst2, ((0, 0), (0, pad))).reshape(-1, CHUNK)
    dst_s = jnp.pad(dst2, ((0, 0), (0, pad)),
                    constant_values=N_NODES).reshape(-1, CHUNK)
    eids = jnp.pad(
        jnp.arange(N_EDGES, dtype=jnp.int32).reshape(NW, EDGES_PER_WORKER),
        ((0, 0), (0, pad))).reshape(-1, CHUNK)
    idx4 = jnp.stack([src, dst_g, eids, dst_s], axis=1).reshape(-1)
    idx4 = jnp.pad(idx4, (0, 2 * 4 * CHUNK))

    we_heads = [jnp.stack([gat[cv][i]['We'] for i in range(HEADS)])
                for cv in range(2)]
    z0, zqk0 = _node_transform(h, w_heads[0], wqkv_heads[0])
    e0 = _edge_transform(e_f, wqkv_heads[0], we_heads[0])
    e1 = _edge_transform(e_f, wqkv_heads[1], we_heads[1])

    def split_s(asum):
        t = asum[0] + asum[1]
        return (t[0:N_NODES].reshape(N_NODES, 1),
                t[N_PAD:N_PAD + N_NODES].reshape(N_NODES, 1),
                t[2 * N_PAD:2 * N_PAD + N_NODES].reshape(N_NODES, 1))

    am0, as0 = _sc_conv(z0, zqk0, e0, idx4)
    cnt0, w00, w10 = split_s(as0)
    h1, z1, zqk1 = _epilogue_mid(am0, cnt0, w00, w10, z0, g[0], b[0],
                                 outs[0]['W'], outs[0]['b'],
                                 w_heads[1], wqkv_heads[1])

    am1, as1 = _sc_conv(z1, zqk1, e1, idx4)
    cnt1, w01, w11 = split_s(as1)
    h2 = _epilogue_last(am1, cnt1, w01, w11,
                        z1, g[1], b[1], outs[1]['W'], outs[1]['b'])

    return _gru_readout(h, h1, h2, params['proj']['W'], params['proj']['b'],
                        params['gru'])


# speedup vs baseline: 1.0077x; 1.0077x over previous
import functools

import jax
import jax.numpy as jnp
from jax import lax
from jax.experimental import pallas as pl
from jax.experimental.pallas import tpu as pltpu
from jax.experimental.pallas import tpu_sc as plsc

N_NODES = 10000
N_EDGES = 160000
IN_FEATS = 74
HIDDEN = 64
HEADS = 2
E_DIM = 11

NC = 2
NS = 16
NW = NC * NS
EDGES_PER_WORKER = N_EDGES // NW
CHUNK = 16
EPW_PAD = 5024
NCHUNK = EPW_PAD // CHUNK
D_NODE = HEADS * 3 * HIDDEN
D_SRC = D_NODE
D_QK = HEADS * 2 * HIDDEN
D_QKD = D_QK
D_OUT = 128
N_PAD = 10240
ROWS_PER_TILE = N_PAD // NS
F32 = jnp.float32


def _node_kernel(h_ref, w_ref, wqkv_ref, z_ref, zqk_ref):
    hb = h_ref[...]
    for hd in range(HEADS):
        t = lax.dot_general(hb, w_ref[hd], (((1,), (1,)), ((), ())),
                            preferred_element_type=F32)
        z = lax.dot_general(t, wqkv_ref[hd], (((1,), (1,)), ((), ())),
                            preferred_element_type=F32)
        z_ref[:, hd * 192:(hd + 1) * 192] = z
        zqk_ref[:, hd * 128:(hd + 1) * 128] = z[:, :128]


def _node_transform(hmat, w_heads, wqkv_heads):
    in_f = hmat.shape[1]
    bn = 2000
    return pl.pallas_call(
        _node_kernel,
        grid=(N_NODES // bn,),
        in_specs=[
            pl.BlockSpec((bn, in_f), lambda i: (i, 0)),
            pl.BlockSpec((HEADS, HIDDEN, in_f), lambda i: (0, 0, 0)),
            pl.BlockSpec((HEADS, 3 * HIDDEN, HIDDEN), lambda i: (0, 0, 0)),
        ],
        out_specs=[
            pl.BlockSpec((bn, D_SRC), lambda i: (i, 0)),
            pl.BlockSpec((bn, D_QKD), lambda i: (i, 0)),
        ],
        out_shape=[
            jax.ShapeDtypeStruct((N_NODES, D_SRC), F32),
            jax.ShapeDtypeStruct((N_NODES, D_QKD), F32),
        ],
    )(hmat, w_heads, wqkv_heads)


def _edge_kernel(ef_ref, wqkv_ref, we_ref, out_ref):
    ef = ef_ref[...]
    for hd in range(HEADS):
        m = lax.dot_general(wqkv_ref[hd], we_ref[hd],
                            (((1,), (0,)), ((), ())),
                            preferred_element_type=F32)
        out = lax.dot_general(ef, m, (((1,), (1,)), ((), ())),
                              preferred_element_type=F32)
        out_ref[:, hd * 192:(hd + 1) * 192] = out


def _edge_transform(e_f, wqkv_heads, we_heads):
    be = 4000
    return pl.pallas_call(
        _edge_kernel,
        grid=(N_EDGES // be,),
        in_specs=[
            pl.BlockSpec((be, E_DIM), lambda i: (i, 0)),
            pl.BlockSpec((HEADS, 3 * HIDDEN, HIDDEN), lambda i: (0, 0, 0)),
            pl.BlockSpec((HEADS, HIDDEN, E_DIM), lambda i: (0, 0, 0)),
        ],
        out_specs=pl.BlockSpec((be, D_SRC), lambda i: (i, 0)),
        out_shape=jax.ShapeDtypeStruct((N_EDGES, D_SRC), F32),
    )(e_f, wqkv_heads, we_heads)


def _sc_body(znode, zqk, eedge, idx4,
             out_m, out_s,
             idxq0, idxq1, isg0, isg1, idg0, idg1, ieg0, ieg1, iw0, iw1,
             iw30, iw31, sr0, sr1, dr0, dr1, er0, er1, mb0, mb1, s30, s31,
             zbuf, zbuf_c, acc_m, acc_s,
             g0, g1, sc0, sc1, ix0, ix1):
    c = lax.axis_index("c")
    s = lax.axis_index("s")

    idxq = (idxq0, idxq1)
    isg = (isg0, isg1)
    idg = (idg0, idg1)
    ieg = (ieg0, ieg1)
    iw = (iw0, iw1)
    iw3 = (iw30, iw31)
    srows = (sr0, sr1)
    drows = (dr0, dr1)
    erows = (er0, er1)
    mbuf = (mb0, mb1)
    s3 = (s30, s31)
    gsem = (g0, g1)
    scsem = (sc0, sc1)
    ixsem = (ix0, ix1)

    zero16 = jnp.zeros((16,), F32)
    one16 = jnp.ones((16,), F32)
    iota16 = lax.broadcasted_iota(jnp.int32, (16,), 0)
    for r in range(zbuf.shape[0]):
        for k in range(D_OUT // 16):
            zbuf[r, pl.ds(k * 16, 16)] = zero16
    for r in range(ROWS_PER_TILE // 16):
        zbuf_c[pl.ds(r * 16, 16)] = zero16
    s30[pl.ds(0, 16)] = one16
    s31[pl.ds(0, 16)] = one16

    row0 = s * ROWS_PER_TILE
    zb = zbuf.shape[0]
    for i in range(ROWS_PER_TILE // zb):
        pltpu.sync_copy(zbuf, acc_m.at[pl.ds(row0 + i * zb, zb)])
    for j in range(3):
        pltpu.sync_copy(zbuf_c, acc_s.at[pl.ds(j * N_PAD + row0,
                                               ROWS_PER_TILE)])
    plsc.subcore_barrier()

    bfly_idx = [jnp.bitwise_xor(iota16, sh) for sh in (8, 4, 2, 1)]
    gdn = lax.GatherDimensionNumbers(offset_dims=(), collapsed_slice_dims=(0,),
                                     start_index_map=(0,))

    def lanesum(v):
        for idx in bfly_idx:
            v = v + lax.gather(v, idx[:, None], gdn, slice_sizes=(1,),
                               mode=lax.GatherScatterMode.PROMISE_IN_BOUNDS)
        return v

    base4 = (c * NS + s) * (EPW_PAD * 4)

    def idx_args(t2, b):
        return (idx4.at[pl.ds(base4 + t2 * (CHUNK * 4), CHUNK * 4)],
                idxq[b], ixsem[b])

    def issue_idx(t2, b):
        pltpu.async_copy(*idx_args(t2, b))

    def wait_idx(t2, b):
        pltpu.make_async_copy(*idx_args(t2, b)).wait()

    def unpack_gidx(b):
        isg[b][pl.ds(0, 16)] = idxq[b][pl.ds(0, 16)]
        idg[b][pl.ds(0, 16)] = idxq[b][pl.ds(16, 16)]
        ieg[b][pl.ds(0, 16)] = idxq[b][pl.ds(32, 16)]

    def unpack_widx(b):
        v = idxq[b][pl.ds(48, 16)]
        iw[b][pl.ds(0, 16)] = v
        iw3[b][pl.ds(0, 16)] = v
        iw3[b][pl.ds(16, 16)] = v + N_PAD
        iw3[b][pl.ds(32, 16)] = v + 2 * N_PAD

    def gather_args(b):
        return ((znode.at[isg[b]], srows[b], gsem[b]),
                (zqk.at[idg[b]], drows[b], gsem[b]),
                (eedge.at[ieg[b]], erows[b], gsem[b]))

    def issue_gathers(b):
        for a in gather_args(b):
            pltpu.async_copy(*a)

    def wait_gathers(b):
        for a in gather_args(b):
            pltpu.make_async_copy(*a).wait()

    def scatter_args(b):
        return ((mbuf[b], acc_m.at[iw[b]], scsem[b]),
                (s3[b], acc_s.at[iw3[b]], scsem[b]))

    def issue_scatters(b):
        for a in scatter_args(b):
            pltpu.async_copy(*a, add=True)

    def wait_scatters(b):
        for a in scatter_args(b):
            pltpu.make_async_copy(*a).wait()

    def compute(b):
        sb, db, eb_, mb = srows[b], drows[b], erows[b], mbuf[b]

        def one_edge(e, carry2):
            c1s = []
            for hd in range(HEADS):
                o = hd * 192
                oq = hd * 128
                qs = [sb[e, pl.ds(o + k * 16, 16)] for k in range(4)]
                ks_ = [sb[e, pl.ds(o + 64 + k * 16, 16)] for k in range(4)]
                vs = [sb[e, pl.ds(o + 128 + k * 16, 16)] for k in range(4)]
                qd = [db[e, pl.ds(oq + k * 16, 16)] for k in range(4)]
                kd = [db[e, pl.ds(oq + 64 + k * 16, 16)] for k in range(4)]
                qe = [eb_[e, pl.ds(o + k * 16, 16)] for k in range(4)]
                ke = [eb_[e, pl.ds(o + 64 + k * 16, 16)] for k in range(4)]
                ve = [eb_[e, pl.ds(o + 128 + k * 16, 16)] for k in range(4)]

                def dotv(a, b2):
                    p = a[0] * b2[0]
                    for k in range(1, 4):
                        p = p + a[k] * b2[k]
                    return lanesum(p)

                lg = [[dotv(qs, ks_), dotv(qs, kd), dotv(qs, ke)],
                      [dotv(qd, ks_), dotv(qd, kd), dotv(qd, ke)],
                      [dotv(qe, ks_), dotv(qe, kd), dotv(qe, ke)]]
                col = [None, None, None]
                for i in range(3):
                    e0 = jnp.exp(lg[i][0])
                    e1 = jnp.exp(lg[i][1])
                    e2 = jnp.exp(lg[i][2])
                    rinv = 1.0 / (e0 + e1 + e2)
                    row = (e0 * rinv, e1 * rinv, e2 * rinv)
                    for j in range(3):
                        col[j] = row[j] if col[j] is None else col[j] + row[j]
                for k in range(4):
                    mk = col[0] * vs[k] + col[2] * ve[k]
                    mb[e, pl.ds(hd * 64 + k * 16, 16)] = mk
                c1s.append(col[1])
            sel = iota16 == jnp.full((16,), e, jnp.int32)
            c1v0 = jnp.where(sel, c1s[0], carry2[0])
            c1v1 = jnp.where(sel, c1s[1], carry2[1])
            return (c1v0, c1v1)

        def group_body(gi, carry2):
            carry2 = one_edge(2 * gi, carry2)
            carry2 = one_edge(2 * gi + 1, carry2)
            return carry2

        final = lax.fori_loop(0, CHUNK // 2, group_body, (zero16, zero16))
        s3[b][pl.ds(16, 16)] = final[0]
        s3[b][pl.ds(32, 16)] = final[1]

    issue_idx(0, 0)
    wait_idx(0, 0)
    unpack_gidx(0)
    unpack_widx(0)
    issue_idx(1, 1)
    wait_idx(1, 1)
    unpack_gidx(1)
    unpack_widx(1)
    issue_gathers(0)
    issue_gathers(1)
    wait_gathers(0)
    issue_idx(2, 0)
    compute(0)
    issue_scatters(0)
    wait_gathers(1)
    issue_idx(3, 1)
    wait_idx(2, 0)
    unpack_gidx(0)
    issue_gathers(0)
    compute(1)
    issue_scatters(1)

    def pair_body(u, carry):
        for b in (0, 1):
            t = 2 * u + b
            wait_gathers(b)
            wait_scatters(b)
            unpack_widx(b)
            issue_idx(t + 2, b)
            wait_idx(t + 1, 1 - b)
            unpack_gidx(1 - b)
            issue_gathers(1 - b)
            compute(b)
            issue_scatters(b)
        return carry

    lax.fori_loop(1, NCHUNK // 2, pair_body, 0)

    wait_scatters(0)
    wait_scatters(1)
    wait_gathers(0)
    wait_idx(NCHUNK + 1, 1)

    plsc.subcore_barrier()
    pltpu.sync_copy(acc_m.at[pl.ds(row0, ROWS_PER_TILE)],
                    out_m.at[c, pl.ds(row0, ROWS_PER_TILE)])
    pltpu.sync_copy(acc_s.at[pl.ds(s * (3 * ROWS_PER_TILE),
                                   3 * ROWS_PER_TILE)],
                    out_s.at[c, pl.ds(s * (3 * ROWS_PER_TILE),
                                      3 * ROWS_PER_TILE)])


_sc_conv = functools.partial(
    pl.kernel,
    out_type=(jax.ShapeDtypeStruct((NC, N_PAD, D_OUT), F32),
              jax.ShapeDtypeStruct((NC, 3 * N_PAD), F32)),
    mesh=plsc.VectorSubcoreMesh(core_axis_name="c", subcore_axis_name="s",
                                num_cores=NC, num_subcores=NS),
    scratch_types=[
        pltpu.VMEM((4 * CHUNK,), jnp.int32),
        pltpu.VMEM((4 * CHUNK,), jnp.int32),
        pltpu.VMEM((CHUNK,), jnp.int32),
        pltpu.VMEM((CHUNK,), jnp.int32),
        pltpu.VMEM((CHUNK,), jnp.int32),
        pltpu.VMEM((CHUNK,), jnp.int32),
        pltpu.VMEM((CHUNK,), jnp.int32),
        pltpu.VMEM((CHUNK,), jnp.int32),
        pltpu.VMEM((CHUNK,), jnp.int32),
        pltpu.VMEM((CHUNK,), jnp.int32),
        pltpu.VMEM((3 * CHUNK,), jnp.int32),
        pltpu.VMEM((3 * CHUNK,), jnp.int32),
        pltpu.VMEM((CHUNK, D_SRC), F32),
        pltpu.VMEM((CHUNK, D_SRC), F32),
        pltpu.VMEM((CHUNK, D_QKD), F32),
        pltpu.VMEM((CHUNK, D_QKD), F32),
        pltpu.VMEM((CHUNK, D_SRC), F32),
        pltpu.VMEM((CHUNK, D_SRC), F32),
        pltpu.VMEM((CHUNK, D_OUT), F32),
        pltpu.VMEM((CHUNK, D_OUT), F32),
        pltpu.VMEM((3 * CHUNK,), F32),
        pltpu.VMEM((3 * CHUNK,), F32),
        pltpu.VMEM((8, D_OUT), F32),
        pltpu.VMEM((ROWS_PER_TILE,), F32),
        pltpu.VMEM_SHARED((N_PAD, D_OUT), F32),
        pltpu.VMEM_SHARED((3 * N_PAD,), F32),
        pltpu.SemaphoreType.DMA,
        pltpu.SemaphoreType.DMA,
        pltpu.SemaphoreType.DMA,
        pltpu.SemaphoreType.DMA,
        pltpu.SemaphoreType.DMA,
        pltpu.SemaphoreType.DMA,
    ],
)(_sc_body)


def _epi_common(acc_ref, cnt_ref, w0_ref, w1_ref, z_ref,
                g_ref, b_ref, wout_ref, bout_ref):
    a = acc_ref[0] + acc_ref[1]
    cnt = cnt_ref[...]
    wh = (w0_ref[...], w1_ref[...])
    cols = []
    for hd in range(HEADS):
        zv = z_ref[:, hd * 192 + 128:hd * 192 + 192]
        m = a[:, hd * 64:(hd + 1) * 64] + wh[hd] * zv
        hn = jnp.where(cnt > 0, m / jnp.maximum(cnt, 1.0), 0.0)
        hn = jnp.maximum(hn, 0.0)
        mu = jnp.mean(hn, axis=-1, keepdims=True)
        var = jnp.mean((hn - mu) ** 2, axis=-1, keepdims=True)
        hn = (hn - mu) * lax.rsqrt(var + 1e-5) * g_ref[hd] + b_ref[hd]
        cols.append(hn)
    hc = jnp.concatenate(cols, axis=1)
    return lax.dot_general(hc, wout_ref[...], (((1,), (1,)), ((), ())),
                           preferred_element_type=F32) + bout_ref[...]


def _epi_kernel_mid(acc_ref, cnt_ref, w0_ref, w1_ref, z_ref,
                    g_ref, b_ref, wout_ref, bout_ref,
                    w1n_ref, wqkv1_ref, h1_ref, z1_ref, zqk1_ref):
    h1 = _epi_common(acc_ref, cnt_ref, w0_ref, w1_ref, z_ref,
                     g_ref, b_ref, wout_ref, bout_ref)
    h1_ref[...] = h1
    for hd in range(HEADS):
        t = lax.dot_general(h1, w1n_ref[hd], (((1,), (1,)), ((), ())),
                            preferred_element_type=F32)
        z = lax.dot_general(t, wqkv1_ref[hd], (((1,), (1,)), ((), ())),
                            preferred_element_type=F32)
        z1_ref[:, hd * 192:(hd + 1) * 192] = z
        zqk1_ref[:, hd * 128:(hd + 1) * 128] = z[:, :128]


def _epi_kernel_last(acc_ref, cnt_ref, w0_ref, w1_ref, z_ref,
                     g_ref, b_ref, wout_ref, bout_ref, h2_ref):
    h2_ref[...] = _epi_common(acc_ref, cnt_ref, w0_ref, w1_ref, z_ref,
                              g_ref, b_ref, wout_ref, bout_ref)


def _epi_in_specs(bn):
    return [
        pl.BlockSpec((NC, bn, D_OUT), lambda i: (0, i, 0)),
        pl.BlockSpec((bn, 1), lambda i: (i, 0)),
        pl.BlockSpec((bn, 1), lambda i: (i, 0)),
        pl.BlockSpec((bn, 1), lambda i: (i, 0)),
        pl.BlockSpec((bn, D_SRC), lambda i: (i, 0)),
        pl.BlockSpec((HEADS, HIDDEN), lambda i: (0, 0)),
        pl.BlockSpec((HEADS, HIDDEN), lambda i: (0, 0)),
        pl.BlockSpec((HIDDEN, HEADS * HIDDEN), lambda i: (0, 0)),
        pl.BlockSpec((HIDDEN,), lambda i: (0,)),
    ]


def _epilogue_mid(acc, cnt, w0, w1, z, g, b, wout, bout, w1n, wqkv1):
    bn = 2000
    return pl.pallas_call(
        _epi_kernel_mid,
        grid=(N_NODES // bn,),
        in_specs=_epi_in_specs(bn) + [
            pl.BlockSpec((HEADS, HIDDEN, HIDDEN), lambda i: (0, 0, 0)),
            pl.BlockSpec((HEADS, 3 * HIDDEN, HIDDEN), lambda i: (0, 0, 0)),
        ],
        out_specs=[
            pl.BlockSpec((bn, HIDDEN), lambda i: (i, 0)),
            pl.BlockSpec((bn, D_SRC), lambda i: (i, 0)),
            pl.BlockSpec((bn, D_QKD), lambda i: (i, 0)),
        ],
        out_shape=[
            jax.ShapeDtypeStruct((N_NODES, HIDDEN), F32),
            jax.ShapeDtypeStruct((N_NODES, D_SRC), F32),
            jax.ShapeDtypeStruct((N_NODES, D_QKD), F32),
        ],
    )(acc, cnt, w0, w1, z, g, b, wout, bout, w1n, wqkv1)


def _epilogue_last(acc, cnt, w0, w1, z, g, b, wout, bout):
    bn = 2000
    return pl.pallas_call(
        _epi_kernel_last,
        grid=(N_NODES // bn,),
        in_specs=_epi_in_specs(bn),
        out_specs=pl.BlockSpec((bn, HIDDEN), lambda i: (i, 0)),
        out_shape=jax.ShapeDtypeStruct((N_NODES, HIDDEN), F32),
    )(acc, cnt, w0, w1, z, g, b, wout, bout)


def _gru_cell(x, hprev, wih, whh, bih, bhh):
    gi = lax.dot_general(x, wih, (((1,), (1,)), ((), ())),
                         preferred_element_type=F32) + bih
    gh = lax.dot_general(hprev, whh, (((1,), (1,)), ((), ())),
                         preferred_element_type=F32) + bhh
    ir, iz, i_n = gi[:, :64], gi[:, 64:128], gi[:, 128:]
    hr, hz, h_n = gh[:, :64], gh[:, 64:128], gh[:, 128:]
    r = jax.nn.sigmoid(ir + hr)
    zz = jax.nn.sigmoid(iz + hz)
    nn_ = jnp.tanh(i_n + r * h_n)
    return (1.0 - zz) * nn_ + zz * hprev


def _gru_kernel(h_ref, h1_ref, h2_ref, pw_ref, pb_ref,
                wih0_ref, whh0_ref, bih0_ref, bhh0_ref,
                wih1_ref, whh1_ref, bih1_ref, bhh1_ref, out_ref):
    mh = jnp.mean(h_ref[...], axis=0, keepdims=True)
    x0 = lax.dot_general(mh, pw_ref[...], (((1,), (1,)), ((), ())),
                         preferred_element_type=F32) + pb_ref[...]
    x1 = jnp.mean(h1_ref[...], axis=0, keepdims=True)
    x2 = jnp.mean(h2_ref[...], axis=0, keepdims=True)
    seq = [x0, x1, x2]
    layers = [(wih0_ref, whh0_ref, bih0_ref, bhh0_ref),
              (wih1_ref, whh1_ref, bih1_ref, bhh1_ref)]
    finals = []
    for wih, whh, bih, bhh in layers:
        hf = jnp.zeros((1, HIDDEN), F32)
        outs_f = []
        for t in range(3):
            hf = _gru_cell(seq[t], hf, wih[0], whh[0], bih[0], bhh[0])
            outs_f.append(hf)
        hb = jnp.zeros((1, HIDDEN), F32)
        outs_b = [None, None, None]
        for t in (2, 1, 0):
            hb = _gru_cell(seq[t], hb, wih[1], whh[1], bih[1], bhh[1])
            outs_b[t] = hb
        finals += [hf, hb]
        seq = [jnp.concatenate([outs_f[t], outs_b[t]], axis=1) for t in range(3)]
    out_ref[...] = finals[0] + finals[1] + finals[2] + finals[3]


def _gru_readout(h, h1, h2, pw, pb, gru):
    wih0 = jnp.stack([gru[0]['f']['W_ih'], gru[0]['b']['W_ih']])
    whh0 = jnp.stack([gru[0]['f']['W_hh'], gru[0]['b']['W_hh']])
    bih0 = jnp.stack([gru[0]['f']['b_ih'], gru[0]['b']['b_ih']])
    bhh0 = jnp.stack([gru[0]['f']['b_hh'], gru[0]['b']['b_hh']])
    wih1 = jnp.stack([gru[1]['f']['W_ih'], gru[1]['b']['W_ih']])
    whh1 = jnp.stack([gru[1]['f']['W_hh'], gru[1]['b']['W_hh']])
    bih1 = jnp.stack([gru[1]['f']['b_ih'], gru[1]['b']['b_ih']])
    bhh1 = jnp.stack([gru[1]['f']['b_hh'], gru[1]['b']['b_hh']])
    whole = lambda *shape: pl.BlockSpec(shape, lambda: tuple(0 for _ in shape))
    return pl.pallas_call(
        _gru_kernel,
        in_specs=[
            whole(N_NODES, IN_FEATS),
            whole(N_NODES, HIDDEN),
            whole(N_NODES, HIDDEN),
            whole(HIDDEN, IN_FEATS),
            whole(HIDDEN,),
            whole(2, 3 * HIDDEN, HIDDEN),
            whole(2, 3 * HIDDEN, HIDDEN),
            whole(2, 3 * HIDDEN),
            whole(2, 3 * HIDDEN),
            whole(2, 3 * HIDDEN, 2 * HIDDEN),
            whole(2, 3 * HIDDEN, HIDDEN),
            whole(2, 3 * HIDDEN),
            whole(2, 3 * HIDDEN),
        ],
        out_specs=whole(1, HIDDEN),
        out_shape=jax.ShapeDtypeStruct((1, HIDDEN), F32),
    )(h, h1, h2, pw, pb, wih0, whh0, bih0, bhh0, wih1, whh1, bih1, bhh1)


def _col(x):
    return (x[0] + x[1])[:N_NODES].reshape(N_NODES, 1)


def kernel(h, e_f, edge_index, params):
    gat = params['gat']
    outs = params['out']

    w_heads = [jnp.stack([gat[cv][i]['W'] for i in range(HEADS)])
               for cv in range(2)]
    wqkv_heads = [
        jnp.stack([jnp.concatenate(
            [gat[cv][i]['Wq'], gat[cv][i]['Wk'], gat[cv][i]['Wv']], axis=0)
            for i in range(HEADS)])
        for cv in range(2)]
    g = [jnp.stack([gat[cv][i]['g'] for i in range(HEADS)]) for cv in range(2)]
    b = [jnp.stack([gat[cv][i]['b'] for i in range(HEADS)]) for cv in range(2)]

    pad = EPW_PAD - EDGES_PER_WORKER
    src = jnp.pad(edge_index[0].reshape(NW, EDGES_PER_WORKER),
                  ((0, 0), (0, pad))).reshape(-1, CHUNK)
    dst2 = edge_index[1].reshape(NW, EDGES_PER_WORKER)
    dst_g = jnp.pad(dst2, ((0, 0), (0, pad))).reshape(-1, CHUNK)
    dst_s = jnp.pad(dst2, ((0, 0), (0, pad)),
                    constant_values=N_NODES).reshape(-1, CHUNK)
    eids = jnp.pad(
        jnp.arange(N_EDGES, dtype=jnp.int32).reshape(NW, EDGES_PER_WORKER),
        ((0, 0), (0, pad))).reshape(-1, CHUNK)
    idx4 = jnp.stack([src, dst_g, eids, dst_s], axis=1).reshape(-1)
    idx4 = jnp.pad(idx4, (0, 2 * 4 * CHUNK))

    we_heads = [jnp.stack([gat[cv][i]['We'] for i in range(HEADS)])
                for cv in range(2)]
    z0, zqk0 = _node_transform(h, w_heads[0], wqkv_heads[0])
    e0 = _edge_transform(e_f, wqkv_heads[0], we_heads[0])
    e1 = _edge_transform(e_f, wqkv_heads[1], we_heads[1])

    def split_s(asum):
        t = asum[0] + asum[1]
        return (t[0:N_NODES].reshape(N_NODES, 1),
                t[N_PAD:N_PAD + N_NODES].reshape(N_NODES, 1),
                t[2 * N_PAD:2 * N_PAD + N_NODES].reshape(N_NODES, 1))

    am0, as0 = _sc_conv(z0, zqk0, e0, idx4)
    cnt0, w00, w10 = split_s(as0)
    h1, z1, zqk1 = _epilogue_mid(am0, cnt0, w00, w10, z0, g[0], b[0],
                                 outs[0]['W'], outs[0]['b'],
                                 w_heads[1], wqkv_heads[1])

    am1, as1 = _sc_conv(z1, zqk1, e1, idx4)
    cnt1, w01, w11 = split_s(as1)
    h2 = _epilogue_last(am1, cnt1, w01, w11,
                        z1, g[1], b[1], outs[1]['W'], outs[1]['b'])

    return _gru_readout(h, h1, h2, params['proj']['W'], params['proj']['b'],
                        params['gru'])
